# Initial kernel scaffold; baseline (speedup 1.0000x reference)
import jax
import jax.numpy as jnp
from jax.experimental import pallas as pl


def kernel(x, edge_index, edge_attr, u, batch, W1a, b1a, g1a, be1a, W1b, b1b, g1b, be1b, W2a, b2a, g2a, be2a, W2b, b2b, g2b, be2b):
    raise NotImplementedError("write your pallas kernel here")


# speedup vs baseline: 2.2873x; 2.2873x over previous
import functools

import jax
import jax.numpy as jnp
from jax import lax
from jax.experimental import pallas as pl
from jax.experimental.pallas import tpu as pltpu
from jax.experimental.pallas import tpu_sc as plsc

_EPS = 1e-5
_NC = 2
_NS = 16


def _sc_gather(x_pad, row, col, npad):
    n, dp = x_pad.shape
    e = row.shape[0]
    nw = _NC * _NS
    per_w = e // nw
    ch = 80
    n_chunks = per_w // ch
    rpt = npad // _NS
    mesh = plsc.VectorSubcoreMesh(core_axis_name="c", subcore_axis_name="s")

    zeros_h = jnp.zeros((npad, 128), jnp.float32)
    ones_h = jnp.ones((ch, 128), jnp.float32)

    @functools.partial(
        pl.kernel,
        out_type=[
            jax.ShapeDtypeStruct((e, dp), jnp.float32),
            jax.ShapeDtypeStruct((npad, 128), jnp.float32),
            jax.ShapeDtypeStruct((npad, 128), jnp.float32),
        ],
        mesh=mesh,
        scratch_types=[
            pltpu.VMEM_SHARED((npad, 128), jnp.float32),
            pltpu.VMEM((ch,), jnp.int32),
            pltpu.VMEM((ch,), jnp.int32),
            pltpu.VMEM((ch, dp), jnp.float32),
            pltpu.VMEM((ch, 128), jnp.float32),
            pltpu.SemaphoreType.DMA,
        ],
    )
    def gather_kernel(x_hbm, row_hbm, col_hbm, zh_hbm, ones_hbm,
                      out_hbm, cnta_hbm, cntb_hbm,
                      cnt_sp, idx_v, cidx_v, rows_v, ones_v, sem):
        c = lax.axis_index("c")
        s = lax.axis_index("s")
        r0 = pl.multiple_of(s * rpt, 8)
        pltpu.sync_copy(zh_hbm.at[pl.ds(r0, rpt)], cnt_sp.at[pl.ds(r0, rpt)])
        pltpu.sync_copy(ones_hbm, ones_v)
        plsc.subcore_barrier()
        base = (s * _NC + c) * per_w

        def chunk(i, carry):
            off = pl.multiple_of(base + i * ch, 8)
            pltpu.sync_copy(row_hbm.at[pl.ds(off, ch)], idx_v)
            pltpu.async_copy(x_hbm.at[idx_v], rows_v, sem).wait()
            pltpu.sync_copy(rows_v, out_hbm.at[pl.ds(off, ch)])
            pltpu.sync_copy(col_hbm.at[pl.ds(off, ch)], cidx_v)
            pltpu.sync_copy(ones_v, cnt_sp.at[cidx_v], add=True)
            return carry

        lax.fori_loop(0, n_chunks, chunk, 0)
        plsc.subcore_barrier()

        @pl.when(c == 0)
        def _():
            pltpu.sync_copy(cnt_sp.at[pl.ds(r0, rpt)], cnta_hbm.at[pl.ds(r0, rpt)])

        @pl.when(c == 1)
        def _():
            pltpu.sync_copy(cnt_sp.at[pl.ds(r0, rpt)], cntb_hbm.at[pl.ds(r0, rpt)])

    return gather_kernel(x_pad, row, col, zeros_h, ones_h)


def _sc_scatter(h2a, h2b, col, n):
    e = col.shape[0]
    hw = h2a.shape[1]
    per_t = e // _NS
    ch = 80
    n_chunks = per_t // ch
    rpt = n // _NS
    mesh = plsc.VectorSubcoreMesh(core_axis_name="c", subcore_axis_name="s")

    zeros_h = jnp.zeros((n, hw), jnp.float32)

    @functools.partial(
        pl.kernel,
        out_type=[
            jax.ShapeDtypeStruct((n, hw), jnp.float32),
            jax.ShapeDtypeStruct((n, hw), jnp.float32),
        ],
        mesh=mesh,
        scratch_types=[
            pltpu.VMEM_SHARED((n, hw), jnp.float32),
            pltpu.VMEM((ch,), jnp.int32),
            pltpu.VMEM((ch, hw), jnp.float32),
        ],
    )
    def scatter_kernel(h2a_hbm, h2b_hbm, col_hbm, zh_hbm,
                       sa_hbm, sb_hbm,
                       s_sp, idx_v, hbuf):
        c = lax.axis_index("c")
        s = lax.axis_index("s")
        r0 = pl.multiple_of(s * rpt, 8)
        pltpu.sync_copy(zh_hbm.at[pl.ds(r0, rpt)], s_sp.at[pl.ds(r0, rpt)])
        plsc.subcore_barrier()

        base = s * per_t

        def do_edges(h2_hbm):
            def chunk(i, carry):
                off = pl.multiple_of(base + i * ch, 8)
                pltpu.sync_copy(col_hbm.at[pl.ds(off, ch)], idx_v)
                pltpu.sync_copy(h2_hbm.at[pl.ds(off, ch)], hbuf)
                pltpu.sync_copy(hbuf, s_sp.at[idx_v], add=True)
                return carry
            lax.fori_loop(0, n_chunks, chunk, 0)

        @pl.when(c == 0)
        def _():
            do_edges(h2a_hbm)

        @pl.when(c == 1)
        def _():
            do_edges(h2b_hbm)

        plsc.subcore_barrier()

        @pl.when(c == 0)
        def _():
            pltpu.sync_copy(s_sp.at[pl.ds(r0, rpt)], sa_hbm.at[pl.ds(r0, rpt)])

        @pl.when(c == 1)
        def _():
            pltpu.sync_copy(s_sp.at[pl.ds(r0, rpt)], sb_hbm.at[pl.ds(r0, rpt)])

    return scatter_kernel(h2a, h2b, col, zeros_h)


def _tc_pass1(xg, ea, wx, we, b1a):
    e, dp = xg.shape
    h = we.shape[1]
    blk = 2560
    grid = e // blk

    def body(xg_ref, ea_ref, wx_ref, we_ref, b_ref, h1_ref, stat_ref):
        i = pl.program_id(0)
        acc = jnp.dot(xg_ref[...], wx_ref[...], preferred_element_type=jnp.float32)
        acc += jnp.dot(ea_ref[...], we_ref[...], preferred_element_type=jnp.float32)
        acc += b_ref[...]
        h1_ref[...] = acc
        blk_stat = jnp.concatenate(
            [jnp.sum(acc, axis=0, keepdims=True),
             jnp.sum(acc * acc, axis=0, keepdims=True)], axis=0)

        @pl.when(i == 0)
        def _():
            stat_ref[...] = blk_stat

        @pl.when(i > 0)
        def _():
            stat_ref[...] += blk_stat

    return pl.pallas_call(
        body,
        grid=(grid,),
        in_specs=[
            pl.BlockSpec((blk, dp), lambda i: (i, 0)),
            pl.BlockSpec((blk, h), lambda i: (i, 0)),
            pl.BlockSpec((dp, h), lambda i: (0, 0)),
            pl.BlockSpec((h, h), lambda i: (0, 0)),
            pl.BlockSpec((1, h), lambda i: (0, 0)),
        ],
        out_specs=[
            pl.BlockSpec((blk, h), lambda i: (i, 0)),
            pl.BlockSpec((2, h), lambda i: (0, 0)),
        ],
        out_shape=[
            jax.ShapeDtypeStruct((e, h), jnp.float32),
            jax.ShapeDtypeStruct((2, h), jnp.float32),
        ],
        compiler_params=pltpu.CompilerParams(
            dimension_semantics=("arbitrary",)),
    )(xg, ea, wx, we, b1a)


def _tc_pass2(h1, stat1, g1a, be1a, w1b, b1b):
    e, h = h1.shape
    hw = h // 2
    blk = 2560
    grid = e // blk
    inv_e = 1.0 / e

    def body(h1_ref, st1_ref, g_ref, be_ref, w_ref, b_ref,
             h2a_ref, h2b_ref, stat_ref):
        i = pl.program_id(0)
        mean = st1_ref[0:1, :] * inv_e
        var = st1_ref[1:2, :] * inv_e - mean * mean
        scale = g_ref[...] * lax.rsqrt(var + _EPS)
        shift = be_ref[...] - mean * scale
        a = jnp.maximum(h1_ref[...] * scale + shift, 0.0)
        h2 = jnp.dot(a, w_ref[...], preferred_element_type=jnp.float32)
        h2 += b_ref[...]
        h2a_ref[...] = h2[:, :hw]
        h2b_ref[...] = h2[:, hw:]
        blk_stat = jnp.concatenate(
            [jnp.sum(h2, axis=0, keepdims=True),
             jnp.sum(h2 * h2, axis=0, keepdims=True)], axis=0)

        @pl.when(i == 0)
        def _():
            stat_ref[...] = blk_stat

        @pl.when(i > 0)
        def _():
            stat_ref[...] += blk_stat

    return pl.pallas_call(
        body,
        grid=(grid,),
        in_specs=[
            pl.BlockSpec((blk, h), lambda i: (i, 0)),
            pl.BlockSpec((2, h), lambda i: (0, 0)),
            pl.BlockSpec((1, h), lambda i: (0, 0)),
            pl.BlockSpec((1, h), lambda i: (0, 0)),
            pl.BlockSpec((h, h), lambda i: (0, 0)),
            pl.BlockSpec((1, h), lambda i: (0, 0)),
        ],
        out_specs=[
            pl.BlockSpec((blk, hw), lambda i: (i, 0)),
            pl.BlockSpec((blk, hw), lambda i: (i, 0)),
            pl.BlockSpec((2, h), lambda i: (0, 0)),
        ],
        out_shape=[
            jax.ShapeDtypeStruct((e, hw), jnp.float32),
            jax.ShapeDtypeStruct((e, hw), jnp.float32),
            jax.ShapeDtypeStruct((2, h), jnp.float32),
        ],
        compiler_params=pltpu.CompilerParams(
            dimension_semantics=("arbitrary",)),
    )(h1, stat1, g1a, be1a, w1b, b1b)


def _tc_node(x_pad, sa, sb, cnta, cntb, stat2, n_edges,
             g1b, be1b, w2x, w2agg, b2a, g2a, be2a, w2b, b2b, g2b, be2b):
    n = x_pad.shape[0]
    h = sa.shape[1] * 2
    inv_e = 1.0 / n_edges
    inv_n = 1.0 / n

    def body(x_ref, sa_ref, sb_ref, cnta_ref, cntb_ref, st2_ref,
             g1b_ref, be1b_ref, w2x_ref, w2agg_ref, b2a_ref,
             g2a_ref, be2a_ref, w2b_ref, b2b_ref, g2b_ref, be2b_ref,
             out_ref):
        mean2 = st2_ref[0:1, :] * inv_e
        var2 = st2_ref[1:2, :] * inv_e - mean2 * mean2
        sc2 = g1b_ref[...] * lax.rsqrt(var2 + _EPS)
        sh2 = be1b_ref[...] - mean2 * sc2
        n = x_ref.shape[0]
        cnt = cnta_ref[:n, 0:1] + cntb_ref[:n, 0:1]
        cclip = jnp.maximum(cnt, 1.0)
        summed = jnp.concatenate([sa_ref[:n, :], sb_ref[:n, :]], axis=1)
        agg = (summed / cclip) * sc2 + sh2
        agg = jnp.where(cnt > 0.0, agg, 0.0)

        hh = jnp.dot(x_ref[...], w2x_ref[...], preferred_element_type=jnp.float32)
        hh += jnp.dot(agg, w2agg_ref[...], preferred_element_type=jnp.float32)
        hh += b2a_ref[...]
        m = jnp.mean(hh, axis=0, keepdims=True)
        v = jnp.mean((hh - m) * (hh - m), axis=0, keepdims=True)
        hh = g2a_ref[...] * (hh - m) * lax.rsqrt(v + _EPS) + be2a_ref[...]
        hh = jnp.maximum(hh, 0.0)
        h2 = jnp.dot(hh, w2b_ref[...], preferred_element_type=jnp.float32)
        h2 += b2b_ref[...]
        m2 = jnp.mean(h2, axis=0, keepdims=True)
        v2 = jnp.mean((h2 - m2) * (h2 - m2), axis=0, keepdims=True)
        out_ref[...] = (g2b_ref[...] * (h2 - m2) * lax.rsqrt(v2 + _EPS)
                        + be2b_ref[...])

    return pl.pallas_call(
        body,
        out_shape=jax.ShapeDtypeStruct((n, h), jnp.float32),
        compiler_params=pltpu.CompilerParams(
            vmem_limit_bytes=120 * 1024 * 1024),
    )(x_pad, sa, sb, cnta, cntb, stat2, g1b, be1b, w2x, w2agg, b2a,
      g2a, be2a, w2b, b2b, g2b, be2b)


def kernel(x, edge_index, edge_attr, u, batch,
           W1a, b1a, g1a, be1a, W1b, b1b, g1b, be1b,
           W2a, b2a, g2a, be2a, W2b, b2b, g2b, be2b):
    n, d = x.shape
    e, h = edge_attr.shape
    dp = 128

    row = edge_index[0]
    col = edge_index[1]
    x_pad = jnp.concatenate([x, jnp.zeros((n, dp - d), x.dtype)], axis=1)
    wx = jnp.concatenate([W1a[:d], jnp.zeros((dp - d, h), W1a.dtype)], axis=0)
    we = W1a[d:]
    w2x = jnp.concatenate([W2a[:d], jnp.zeros((dp - d, h), W2a.dtype)], axis=0)
    w2agg = W2a[d:]
    r2 = lambda a: a.reshape(1, h)

    npad = ((n + _NS * 80 - 1) // (_NS * 80)) * (_NS * 80)
    xg, cnta, cntb = _sc_gather(x_pad, row, col, npad)
    h1, stat1 = _tc_pass1(xg, edge_attr, wx, we, r2(b1a))
    h2a, h2b, stat2 = _tc_pass2(h1, stat1, r2(g1a), r2(be1a), W1b, r2(b1b))
    sa, sb = _sc_scatter(h2a, h2b, col, npad)
    return _tc_node(x_pad, sa, sb, cnta, cntb, stat2, float(e),
                    r2(g1b), r2(be1b), w2x, w2agg, r2(b2a),
                    r2(g2a), r2(be2a), W2b, r2(b2b), r2(g2b), r2(be2b))
